# Initial kernel scaffold; baseline (speedup 1.0000x reference)
import jax
import jax.numpy as jnp
from jax.experimental import pallas as pl


def kernel(encoding, x):
    raise NotImplementedError("write your pallas kernel here")


# speedup vs baseline: 6.9955x; 6.9955x over previous
import functools

import jax
import jax.numpy as jnp
from jax import lax
from jax.experimental import pallas as pl
from jax.experimental.pallas import tpu as pltpu
from jax.experimental.pallas import tpu_sc as plsc

B, T, EMB = 4096, 200, 128
N = B * T
NC, NS = 2, 16
NW = NC * NS
PER_W = N // NW
CHUNK = 128
NCHUNK = PER_W // CHUNK


@functools.partial(
    pl.kernel,
    out_type=jax.ShapeDtypeStruct((N, EMB), jnp.float32),
    mesh=plsc.VectorSubcoreMesh(core_axis_name="c", subcore_axis_name="s"),
    scratch_types=[
        pltpu.VMEM((NCHUNK, CHUNK), jnp.int32),
        pltpu.VMEM((CHUNK, EMB), jnp.float32),
        pltpu.SemaphoreType.DMA,
    ],
)
def _sc_gather(table, idx, out, idx_v, rows_v, sem):
    w = lax.axis_index("s") * NC + lax.axis_index("c")
    base = w * PER_W
    pltpu.sync_copy(idx.at[w], idx_v)

    def step(g, carry):
        pltpu.async_copy(table.at[idx_v.at[g]], rows_v, sem).wait()
        pltpu.sync_copy(rows_v, out.at[pl.ds(base + g * CHUNK, CHUNK)])
        return carry

    lax.fori_loop(0, NCHUNK, step, 0)


def kernel(encoding, x):
    idx = x.reshape(N).astype(jnp.int32).reshape(NW, NCHUNK, CHUNK)
    out = _sc_gather(encoding, idx)
    return out.reshape(B, T, EMB)


# speedup vs baseline: 10.0932x; 1.4428x over previous
import functools

import jax
import jax.numpy as jnp
from jax import lax
from jax.experimental import pallas as pl
from jax.experimental.pallas import tpu as pltpu
from jax.experimental.pallas import tpu_sc as plsc

B, T, EMB = 4096, 200, 128
N = B * T
NC, NS = 2, 16
NW = NC * NS
PER_W = N // NW
CHUNK = 128
NCHUNK = PER_W // CHUNK


NBUF = 4


@functools.partial(
    pl.kernel,
    out_type=jax.ShapeDtypeStruct((N, EMB), jnp.float32),
    mesh=plsc.VectorSubcoreMesh(core_axis_name="c", subcore_axis_name="s"),
    scratch_types=[
        pltpu.VMEM((NCHUNK, CHUNK), jnp.int32),
    ]
    + [pltpu.VMEM((CHUNK, EMB), jnp.float32) for _ in range(NBUF)]
    + [pltpu.SemaphoreType.DMA for _ in range(NBUF)],
)
def _sc_gather(table, idx, out, idx_v, r0, r1, r2, r3, s0, s1, s2, s3):
    rows = (r0, r1, r2, r3)
    sems = (s0, s1, s2, s3)
    w = lax.axis_index("s") * NC + lax.axis_index("c")
    base = w * PER_W
    pltpu.sync_copy(idx.at[w], idx_v)

    def g_start(g, b):
        pltpu.async_copy(table.at[idx_v.at[g]], rows[b], sems[b])

    def g_wait(g, b):
        pltpu.make_async_copy(table.at[idx_v.at[g]], rows[b], sems[b]).wait()

    for b in range(NBUF - 1):
        g_start(b, b)

    def body(t, carry):
        for b in range(NBUF):
            g = t * NBUF + b
            g_wait(g, b)
            pltpu.sync_copy(rows[b], out.at[pl.ds(base + g * CHUNK, CHUNK)])
            g_start(g + NBUF - 1, (b + NBUF - 1) % NBUF)
        return carry

    lax.fori_loop(0, NCHUNK // NBUF - 1, body, 0)

    for b in range(NBUF):
        g = NCHUNK - NBUF + b
        g_wait(g, b)
        pltpu.sync_copy(rows[b], out.at[pl.ds(base + g * CHUNK, CHUNK)])
        if b == 0:
            g_start(g + NBUF - 1, (b + NBUF - 1) % NBUF)


def kernel(encoding, x):
    idx = x.reshape(N).astype(jnp.int32).reshape(NW, NCHUNK, CHUNK)
    out = _sc_gather(encoding, idx)
    return out.reshape(B, T, EMB)


# speedup vs baseline: 10.1057x; 1.0012x over previous
import functools

import jax
import jax.numpy as jnp
from jax import lax
from jax.experimental import pallas as pl
from jax.experimental.pallas import tpu as pltpu
from jax.experimental.pallas import tpu_sc as plsc

B, T, EMB = 4096, 200, 128
N = B * T
NC, NS = 2, 16
NW = NC * NS
PER_W = N // NW
CHUNK = 128
NCHUNK = PER_W // CHUNK


NBUF = 5


@functools.partial(
    pl.kernel,
    out_type=jax.ShapeDtypeStruct((N, EMB), jnp.float32),
    mesh=plsc.VectorSubcoreMesh(core_axis_name="c", subcore_axis_name="s"),
    scratch_types=[
        pltpu.VMEM((NCHUNK, CHUNK), jnp.int32),
    ]
    + [pltpu.VMEM((CHUNK, EMB), jnp.float32) for _ in range(NBUF)]
    + [pltpu.SemaphoreType.DMA for _ in range(2 * NBUF)],
)
def _sc_gather(table, idx, out, idx_v, *bufs):
    rows = bufs[:NBUF]
    gsems = bufs[NBUF:2 * NBUF]
    osems = bufs[2 * NBUF:]
    w = lax.axis_index("s") * NC + lax.axis_index("c")
    base = w * PER_W
    pltpu.sync_copy(idx.at[w], idx_v)

    def g_start(g, b):
        pltpu.async_copy(table.at[idx_v.at[g]], rows[b], gsems[b])

    def g_wait(g, b):
        pltpu.make_async_copy(table.at[idx_v.at[g]], rows[b], gsems[b]).wait()

    def o_start(g, b):
        pltpu.async_copy(rows[b], out.at[pl.ds(base + g * CHUNK, CHUNK)], osems[b])

    def o_wait(g, b):
        pltpu.make_async_copy(
            rows[b], out.at[pl.ds(base + g * CHUNK, CHUNK)], osems[b]
        ).wait()

    def chunk_ops(g, b, do_owait, do_gstart):
        if do_owait:
            o_wait(g - 3, (b - 3) % NBUF)
        if do_gstart:
            g_start(g + 2, (b + 2) % NBUF)
        g_wait(g, b)
        o_start(g, b)

    g_start(0, 0)
    g_start(1, 1)
    for b in range(NBUF):
        chunk_ops(b, b, b >= 3, True)

    def body(t, carry):
        for b in range(NBUF):
            chunk_ops(t * NBUF + b, b, True, True)
        return carry

    lax.fori_loop(1, NCHUNK // NBUF - 1, body, 0)

    for b in range(NBUF):
        g = NCHUNK - NBUF + b
        chunk_ops(g, b, True, g + 2 < NCHUNK)
    for g in range(NCHUNK - 3, NCHUNK):
        o_wait(g, g % NBUF)


def kernel(encoding, x):
    idx = x.reshape(N).astype(jnp.int32).reshape(NW, NCHUNK, CHUNK)
    out = _sc_gather(encoding, idx)
    return out.reshape(B, T, EMB)


# speedup vs baseline: 17.6371x; 1.7453x over previous
import functools

import jax
import jax.numpy as jnp
from jax import lax
from jax.experimental import pallas as pl
from jax.experimental.pallas import tpu as pltpu
from jax.experimental.pallas import tpu_sc as plsc

B, T, EMB = 4096, 200, 128
MAX_LEN = 8192
N = B * T
NC, NS = 2, 16
NW = NC * NS
PER_W = N // NW
CHUNK = 64
NCHUNK = PER_W // CHUNK

NBUF = 4
NIDX = 8


@functools.partial(
    pl.kernel,
    out_type=jax.ShapeDtypeStruct((N, EMB), jnp.float32),
    mesh=plsc.VectorSubcoreMesh(core_axis_name="c", subcore_axis_name="s"),
    scratch_types=[
        pltpu.VMEM((NIDX, CHUNK), jnp.int32),
        pltpu.VMEM_SHARED((MAX_LEN, EMB), jnp.float32),
    ]
    + [pltpu.VMEM((CHUNK, EMB), jnp.float32) for _ in range(NBUF)]
    + [pltpu.SemaphoreType.DMA for _ in range(2 * NBUF + NIDX)],
)
def _sc_gather(table, idx, out, idx_v, table_sh, *bufs):
    rows = bufs[:NBUF]
    gsems = bufs[NBUF:2 * NBUF]
    osems = bufs[2 * NBUF:3 * NBUF]
    isems = bufs[3 * NBUF:]
    sid = lax.axis_index("s")
    w = sid * NC + lax.axis_index("c")
    base = w * PER_W

    stripe = MAX_LEN // NS
    pltpu.sync_copy(
        table.at[pl.ds(sid * stripe, stripe)],
        table_sh.at[pl.ds(sid * stripe, stripe)],
    )
    plsc.subcore_barrier()

    def i_start(g, s):
        pltpu.async_copy(idx.at[w, g], idx_v.at[s], isems[s])

    def i_wait(g, s):
        pltpu.make_async_copy(idx.at[w, g], idx_v.at[s], isems[s]).wait()

    def g_start(b, s):
        pltpu.async_copy(table_sh.at[idx_v.at[s]], rows[b], gsems[b])

    def g_wait(b, s):
        pltpu.make_async_copy(table_sh.at[idx_v.at[s]], rows[b], gsems[b]).wait()

    def o_start(g, b):
        pltpu.async_copy(rows[b], out.at[pl.ds(base + g * CHUNK, CHUNK)], osems[b])

    def o_wait(g, b):
        pltpu.make_async_copy(
            rows[b], out.at[pl.ds(base + g * CHUNK, CHUNK)], osems[b]
        ).wait()

    def chunk_ops(g, s, do_owait=True, do_istart=True, do_gstart=True):
        b = s % NBUF
        if do_istart:
            i_start(g + NBUF, (s + NBUF) % NIDX)
        if do_owait:
            o_wait(g - 2, (b - 2) % NBUF)
        if do_gstart:
            i_wait(g + 2, (s + 2) % NIDX)
            g_start((b + 2) % NBUF, (s + 2) % NIDX)
        g_wait(b, s)
        o_start(g, b)

    for g in range(NBUF):
        i_start(g, g)
    for g in range(2):
        i_wait(g, g)
        g_start(g, g)
    for g in range(NIDX):
        chunk_ops(g, g, do_owait=g >= 2)

    def body(t, carry):
        for j in range(NIDX):
            chunk_ops(t * NIDX + j, j)
        return carry

    lax.fori_loop(1, NCHUNK // NIDX - 1, body, 0)

    for j in range(NIDX):
        g = NCHUNK - NIDX + j
        chunk_ops(
            g,
            j,
            do_istart=g + NBUF < NCHUNK,
            do_gstart=g + 2 < NCHUNK,
        )
    for g in range(NCHUNK - 2, NCHUNK):
        o_wait(g, g % NBUF)


def kernel(encoding, x):
    idx = x.reshape(N).astype(jnp.int32).reshape(NW, NCHUNK, CHUNK)
    out = _sc_gather(encoding, idx)
    return out.reshape(B, T, EMB)
